# Initial kernel scaffold; baseline (speedup 1.0000x reference)
import jax
import jax.numpy as jnp
from jax.experimental import pallas as pl


def kernel(h, affine, edge_index):
    raise NotImplementedError("write your pallas kernel here")


# speedup vs baseline: 1.8792x; 1.8792x over previous
import functools

import jax
import jax.numpy as jnp
from jax import lax
from jax.experimental import pallas as pl
from jax.experimental.pallas import tpu as pltpu
from jax.experimental.pallas import tpu_sc as plsc

_NC = 2
_NS = 16
_NW = _NC * _NS
_LANES = 16


@functools.partial(jax.jit, static_argnames=())
def _u_mul_e(h, src, affine):
    E, D = affine.shape
    assert E % _NW == 0
    ew = E // _NW
    C = 80
    assert ew % C == 0
    n_chunks = ew // C
    vregs_per_row = D // _LANES

    mesh = plsc.VectorSubcoreMesh(core_axis_name="c", subcore_axis_name="s")

    @functools.partial(
        pl.kernel,
        mesh=mesh,
        out_type=jax.ShapeDtypeStruct((E, D), jnp.float32),
        scratch_types=[
            pltpu.VMEM((ew,), jnp.int32),
            pltpu.VMEM((C, D), jnp.float32),
            pltpu.VMEM((C, D), jnp.float32),
            pltpu.SemaphoreType.DMA,
        ],
    )
    def run(h_hbm, src_hbm, aff_hbm, out_hbm, idx_v, rows_v, aff_v, sem):
        wid = lax.axis_index("s") * _NC + lax.axis_index("c")
        base_w = wid * ew
        pltpu.sync_copy(src_hbm.at[pl.ds(base_w, ew)], idx_v)

        def chunk(i, carry):
            base = base_w + i * C
            pltpu.async_copy(
                h_hbm.at[idx_v.at[pl.ds(i * C, C)]], rows_v, sem
            ).wait()
            pltpu.sync_copy(aff_hbm.at[pl.ds(base, C)], aff_v)

            def row(r, rc):
                for v in range(vregs_per_row):
                    sl = pl.ds(v * _LANES, _LANES)
                    aff_v[r, sl] = aff_v[r, sl] * rows_v[r, sl]
                return rc

            lax.fori_loop(0, C, row, 0)
            pltpu.sync_copy(aff_v, out_hbm.at[pl.ds(base, C)])
            return carry

        lax.fori_loop(0, n_chunks, chunk, 0)

    return run(h, src, affine)


def kernel(h, affine, edge_index):
    return _u_mul_e(h, edge_index[0], affine)


# speedup vs baseline: 4.2864x; 2.2809x over previous
import functools

import jax
import jax.numpy as jnp
from jax import lax
from jax.experimental import pallas as pl
from jax.experimental.pallas import tpu as pltpu
from jax.experimental.pallas import tpu_sc as plsc

_NC = 2
_NS = 16
_NW = _NC * _NS
_LANES = 16
_NB = 5
_C = 40


@jax.jit
def _u_mul_e(h, src, affine):
    E, D = affine.shape
    assert E % (_NW * _C) == 0
    ew = E // _NW
    n_chunks = ew // _C
    n_groups = n_chunks // _NB
    assert n_chunks % _NB == 0 and n_groups >= 2
    vregs_per_row = D // _LANES

    mesh = plsc.VectorSubcoreMesh(core_axis_name="c", subcore_axis_name="s")

    buf_types = [pltpu.VMEM((_C, D), jnp.float32) for _ in range(3 * _NB)]
    sem_types = [pltpu.SemaphoreType.DMA for _ in range(3 * _NB)]

    @functools.partial(
        pl.kernel,
        mesh=mesh,
        out_type=jax.ShapeDtypeStruct((E, D), jnp.float32),
        scratch_types=[pltpu.VMEM((ew,), jnp.int32)] + buf_types + sem_types,
    )
    def run(h_hbm, src_hbm, aff_hbm, out_hbm, idx_v, *rest):
        rows = rest[0:_NB]
        aff = rest[_NB:2 * _NB]
        outb = rest[2 * _NB:3 * _NB]
        gsem = rest[3 * _NB:4 * _NB]
        asem = rest[4 * _NB:5 * _NB]
        ssem = rest[5 * _NB:6 * _NB]

        wid = lax.axis_index("s") * _NC + lax.axis_index("c")
        base_w = wid * ew
        pltpu.sync_copy(src_hbm.at[pl.ds(base_w, ew)], idx_v)

        def issue_loads(i, b):
            pltpu.async_copy(
                h_hbm.at[idx_v.at[pl.ds(i * _C, _C)]], rows[b], gsem[b])
            pltpu.async_copy(
                aff_hbm.at[pl.ds(base_w + i * _C, _C)], aff[b], asem[b])

        def wait_loads(b):
            pltpu.make_async_copy(
                h_hbm.at[pl.ds(0, _C)], rows[b], gsem[b]).wait()
            pltpu.make_async_copy(
                aff_hbm.at[pl.ds(0, _C)], aff[b], asem[b]).wait()

        def issue_store(i, b):
            pltpu.async_copy(
                outb[b], out_hbm.at[pl.ds(base_w + i * _C, _C)], ssem[b])

        def wait_store(b):
            pltpu.make_async_copy(
                outb[b], out_hbm.at[pl.ds(0, _C)], ssem[b]).wait()

        def compute(b):
            def row(r, rc):
                for v in range(vregs_per_row):
                    sl = pl.ds(v * _LANES, _LANES)
                    outb[b][r, sl] = rows[b][r, sl] * aff[b][r, sl]
                return rc

            lax.fori_loop(0, _C, row, 0)

        for b in range(_NB - 1):
            issue_loads(b, b)

        for b in range(_NB):
            issue_loads(b + _NB - 1, (b + _NB - 1) % _NB)
            wait_loads(b)
            compute(b)
            issue_store(b, b)

        def group(g, carry):
            i0 = g * _NB
            for b in range(_NB):
                issue_loads(i0 + b + _NB - 1, (b + _NB - 1) % _NB)
                wait_loads(b)
                wait_store(b)
                compute(b)
                issue_store(i0 + b, b)
            return carry

        lax.fori_loop(1, n_groups - 1, group, 0)

        i0 = (n_groups - 1) * _NB
        issue_loads(i0 + _NB - 1, (_NB - 1) % _NB)
        for b in range(_NB):
            wait_loads(b)
            wait_store(b)
            compute(b)
            issue_store(i0 + b, b)

        for b in range(_NB):
            wait_store(b)

    return run(h, src, affine)


def kernel(h, affine, edge_index):
    return _u_mul_e(h, edge_index[0], affine)
